# baseline (speedup 1.0000x reference)
import functools

import jax
import jax.numpy as jnp
import numpy as np
from jax.experimental import pallas as pl
from jax.experimental.pallas import tpu as pltpu

_DRN2 = 0.4 * 0.4
_TWO_PI = 2.0 * np.pi


def _leaky(x):
    return jnp.maximum(x, 0.01 * x)


def _bf(x):
    return x.astype(jnp.bfloat16).astype(jnp.float32)


def _contract(cols, f):
    acc = cols[0] * f[0][None]
    for k in range(1, len(cols)):
        acc = acc + cols[k] * f[k][None]
    return acc


def _pair_geom(cx, cy):
    dx = cx[:, :, None] - cx[:, None, :]
    dy = cy[:, :, None] - cy[:, None, :]
    d2 = dx * dx + dy * dy
    adjf = jnp.where(d2 <= _DRN2, 1.0, 0.0).astype(jnp.float32)
    n2 = cx * cx + cy * cy
    norm = jnp.sqrt(n2[:, :, None] * n2[:, None, :])
    cos = (cx[:, :, None] * cx[:, None, :] + cy[:, :, None] * cy[:, None, :]) / norm
    sin = (cy[:, :, None] * cx[:, None, :] - cx[:, :, None] * cy[:, None, :]) / norm
    return adjf, cos, sin


def _conv_layer(load_chunk, W0T, b0, g0, be0, W1T, b1,
                xs_ref, nchunks, F, store_xr=None):
    WiT = _bf(W0T[:, 0:F])
    WjT = _bf(W0T[:, F:2 * F])
    WdT = _bf(W0T[:, 2 * F:3 * F])
    wi_cols = [WiT[:, k][:, None, None] for k in range(F)]
    wj_cols = [WjT[:, k][:, None, None] for k in range(F)]
    wd_cols = [WdT[:, k][:, None, None, None] for k in range(F)]
    wcc = _bf(W0T[:, 3 * F])[:, None, None, None]
    wsc = _bf(W0T[:, 3 * F + 1])[:, None, None, None]
    b0c = b0[:, None, None, None]
    w1_cols = [_bf(W1T)[:, k][:, None, None] for k in range(10)]

    def chunk_core(i):
        xc = load_chunk(i)
        f = xc[1:1 + F]
        adjf, cos, sin = _pair_geom(xc[-2], xc[-1])
        fb = _bf(f)
        A = _contract(wi_cols, fb)
        Bv = _contract(wj_cols, fb)
        fd = _bf(f[:, :, None, :] - f[:, :, :, None])
        hd = wd_cols[0] * fd[0][None]
        for k in range(1, F):
            hd = hd + wd_cols[k] * fd[k][None]
        h = (A[:, :, :, None] + Bv[:, :, None, :] + hd
             + _bf(cos)[None] * wcc + _bf(sin)[None] * wsc + b0c)
        return xc, adjf, h

    bc = xs_ref.shape[2]
    N = xs_ref.shape[3]

    def stats_body(i, carry):
        S1p, S2p, cntp = carry
        _, adjf, h = chunk_core(i)
        hm = h * adjf[None]
        S1p = S1p + hm.sum(axis=3)
        S2p = S2p + (hm * h).sum(axis=3)
        cntp = cntp + adjf.sum(axis=2)
        return S1p, S2p, cntp

    S1p, S2p, cntp = jax.lax.fori_loop(
        0, nchunks, stats_body,
        (jnp.zeros((10, bc, N), jnp.float32),
         jnp.zeros((10, bc, N), jnp.float32),
         jnp.zeros((bc, N), jnp.float32)))
    S1 = S1p.sum(axis=2).sum(axis=1, keepdims=True)
    S2 = S2p.sum(axis=2).sum(axis=1, keepdims=True)
    cnt = cntp.sum(axis=1, keepdims=True).sum(axis=0, keepdims=True)
    cnt = jnp.maximum(cnt, 1.0)
    mean = S1 / cnt
    var = S2 / cnt - mean * mean
    scale = g0[:, None] / jnp.sqrt(var + 1e-5)
    shift = be0[:, None]
    meanc = mean[:, :, None, None]
    scalec = scale[:, :, None, None]
    shiftc = shift[:, :, None, None]
    b1c = b1[:, None, None]

    def apply_body(i, _):
        xc, adjf, h = chunk_core(i)
        hn = (h - meanc) * scalec + shiftc
        r = _bf(_leaky(hn)) * adjf[None]
        rs = r.sum(axis=3)
        deg = adjf.sum(axis=2)
        hmean = rs / deg[None]
        res = _contract(w1_cols, hmean) + b1c
        g = res[10]
        t = g - jnp.round(g)
        c = jnp.cos(_TWO_PI * t)
        s = jnp.sin(_TWO_PI * t)
        zr = xc[-2]
        zi = xc[-1]
        nr = c * zr - s * zi
        ni = s * zr + c * zi
        xn = jnp.concatenate([xc[0:1], res[0:10], nr[None], ni[None]], axis=0)
        xs_ref[i] = xn
        if store_xr is not None:
            store_xr(i, xn)
        return 0

    jax.lax.fori_loop(0, nchunks, apply_body, 0)


def _bn1d(h, gamma, beta):
    mean = h.mean(axis=0)
    var = ((h - mean) ** 2).mean(axis=0)
    return gamma * (h - mean) / jnp.sqrt(var + 1e-5) + beta


def _body(pt_ref, ax_ref, ay_ref,
          W0aT_ref, b0a_ref, g0a_ref, be0a_ref, W1aT_ref, b1a_ref,
          W0bT_ref, b0b_ref, g0b_ref, be0b_ref, W1bT_ref, b1b_ref,
          RW0_ref, Rb0_ref, Rg0_ref, Rbe0_ref,
          RW1_ref, Rb1_ref, Rg1_ref, Rbe1_ref, RW2_ref, Rb2_ref,
          out_ref, xs_ref, xr_ref, *, bc, nchunks):

    def load_a(i):
        ptc = pt_ref[pl.ds(i * bc, bc)]
        axc = ax_ref[pl.ds(i * bc, bc)]
        ayc = ay_ref[pl.ds(i * bc, bc)]
        absang = jnp.sqrt(axc * axc + ayc * ayc)
        return jnp.concatenate(
            [ptc[None], absang[None], axc[None], ayc[None]], axis=0)

    _conv_layer(load_a, W0aT_ref[...], b0a_ref[...], g0a_ref[...],
                be0a_ref[...], W1aT_ref[...], b1a_ref[...],
                xs_ref, nchunks, 1)

    def load_b(i):
        return xs_ref[i]

    def store_xr(i, xn):
        ptc = pt_ref[pl.ds(i * bc, bc)]
        wjc = ptc / ptc.sum(axis=1, keepdims=True)
        xr_c = (wjc[None] * xn[1:13]).sum(axis=2)
        xr_ref[pl.ds(i * bc, bc), :] = jnp.transpose(xr_c, (1, 0))

    _conv_layer(load_b, W0bT_ref[...], b0b_ref[...], g0b_ref[...],
                be0b_ref[...], W1bT_ref[...], b1b_ref[...],
                xs_ref, nchunks, 10, store_xr=store_xr)

    xr = xr_ref[...]
    h = jnp.dot(xr[:, :10].astype(jnp.bfloat16),
                RW0_ref[...].astype(jnp.bfloat16),
                preferred_element_type=jnp.float32) + Rb0_ref[...]
    h = _leaky(_bn1d(h, Rg0_ref[...], Rbe0_ref[...]))
    h = jnp.dot(h.astype(jnp.bfloat16), RW1_ref[...].astype(jnp.bfloat16),
                preferred_element_type=jnp.float32) + Rb1_ref[...]
    h = _leaky(_bn1d(h, Rg1_ref[...], Rbe1_ref[...]))
    h = jnp.dot(h.astype(jnp.bfloat16), RW2_ref[...].astype(jnp.bfloat16),
                preferred_element_type=jnp.float32) + Rb2_ref[...]
    out = 1.0 / (1.0 + jnp.exp(-h))
    out_ref[...] = jnp.concatenate([out, xr[:, 10:12]], axis=-1)


def kernel(pt, angles, W0a, b0a, g0a, be0a, W1a, b1a,
           W0b, b0b, g0b, be0b, W1b, b1b,
           RW0, Rb0, Rg0, Rbe0, RW1, Rb1, Rg1, Rbe1, RW2, Rb2):
    B, N = pt.shape
    bc = 32 if B % 32 == 0 else B
    nchunks = B // bc

    body = functools.partial(_body, bc=bc, nchunks=nchunks)

    return pl.pallas_call(
        body,
        out_shape=jax.ShapeDtypeStruct((B, 3), jnp.float32),
        scratch_shapes=[
            pltpu.VMEM((nchunks, 13, bc, N), jnp.float32),
            pltpu.VMEM((B, 12), jnp.float32),
        ],
    )(pt, angles[..., 0], angles[..., 1],
      W0a.T, b0a, g0a, be0a, W1a.T, b1a,
      W0b.T, b0b, g0b, be0b, W1b.T, b1b,
      RW0, Rb0, Rg0, Rbe0, RW1, Rb1, Rg1, Rbe1, RW2, Rb2)


# speedup vs baseline: 1.0433x; 1.0433x over previous
import functools

import jax
import jax.numpy as jnp
import numpy as np
from jax.experimental import pallas as pl
from jax.experimental.pallas import tpu as pltpu

_DRN2 = 0.4 * 0.4
_TWO_PI = 2.0 * np.pi


def _leaky(x):
    return jnp.maximum(x, 0.01 * x)


def _bf(x):
    return x.astype(jnp.bfloat16).astype(jnp.float32)


def _contract(WT, f):
    F = f.shape[0]
    acc = WT[:, 0][:, None, None] * f[0][None]
    for k in range(1, F):
        acc = acc + WT[:, k][:, None, None] * f[k][None]
    return acc


def _pair_geom(cx, cy):
    dx = cx[:, :, None] - cx[:, None, :]
    dy = cy[:, :, None] - cy[:, None, :]
    d2 = dx * dx + dy * dy
    adjf = jnp.where(d2 <= _DRN2, 1.0, 0.0).astype(jnp.float32)
    n2 = cx * cx + cy * cy
    norm = jnp.sqrt(n2[:, :, None] * n2[:, None, :])
    cos = (cx[:, :, None] * cx[:, None, :] + cy[:, :, None] * cy[:, None, :]) / norm
    sin = (cy[:, :, None] * cx[:, None, :] - cx[:, :, None] * cy[:, None, :]) / norm
    return adjf, cos, sin


def _conv_layer(load_chunk, W0T, b0, g0, be0, W1T, b1,
                xs_ref, nchunks, F, store_xr=None):
    WiT = _bf(W0T[:, 0:F])
    WjT = _bf(W0T[:, F:2 * F])
    WdT = _bf(W0T[:, 2 * F:3 * F])
    wd_cols = [WdT[:, k][:, None, None, None] for k in range(F)]
    wcc = _bf(W0T[:, 3 * F])[:, None, None, None]
    wsc = _bf(W0T[:, 3 * F + 1])[:, None, None, None]
    b0c = b0[:, None, None, None]
    bW1T = _bf(W1T)

    def chunk_core(i):
        xc = load_chunk(i)
        f = xc[1:1 + F]
        adjf, cos, sin = _pair_geom(xc[-2], xc[-1])
        fb = _bf(f)
        A = _contract(WiT, fb)
        Bv = _contract(WjT, fb)
        fd = _bf(f[:, :, None, :] - f[:, :, :, None])
        hd = wd_cols[0] * fd[0][None]
        for k in range(1, F):
            hd = hd + wd_cols[k] * fd[k][None]
        h = (A[:, :, :, None] + Bv[:, :, None, :] + hd
             + _bf(cos)[None] * wcc + _bf(sin)[None] * wsc + b0c)
        return xc, adjf, h

    def stats_body(i, carry):
        S1, S2, cnt = carry
        _, adjf, h = chunk_core(i)
        hm = h * adjf[None]
        S1 = S1 + hm.sum(axis=3).sum(axis=2).sum(axis=1, keepdims=True)
        S2 = S2 + (hm * h).sum(axis=3).sum(axis=2).sum(axis=1, keepdims=True)
        cnt = cnt + adjf.sum(axis=2).sum(axis=1, keepdims=True).sum(
            axis=0, keepdims=True)
        return S1, S2, cnt

    S1, S2, cnt = jax.lax.fori_loop(
        0, nchunks, stats_body,
        (jnp.zeros((10, 1), jnp.float32), jnp.zeros((10, 1), jnp.float32),
         jnp.zeros((1, 1), jnp.float32)))
    cnt = jnp.maximum(cnt, 1.0)
    mean = S1 / cnt
    var = S2 / cnt - mean * mean
    scale = g0[:, None] / jnp.sqrt(var + 1e-5)
    shift = be0[:, None]
    meanc = mean[:, :, None, None]
    scalec = scale[:, :, None, None]
    shiftc = shift[:, :, None, None]
    b1c = b1[:, None, None]

    def apply_body(i, _):
        xc, adjf, h = chunk_core(i)
        hn = (h - meanc) * scalec + shiftc
        r = _bf(_leaky(hn)) * adjf[None]
        rs = r.sum(axis=3)
        deg = adjf.sum(axis=2)
        hmean = rs / deg[None]
        res = _contract(bW1T, hmean) + b1c
        g = res[10]
        t = g - jnp.round(g)
        c = jnp.cos(_TWO_PI * t)
        s = jnp.sin(_TWO_PI * t)
        zr = xc[-2]
        zi = xc[-1]
        nr = c * zr - s * zi
        ni = s * zr + c * zi
        xn = jnp.concatenate([xc[0:1], res[0:10], nr[None], ni[None]], axis=0)
        xs_ref[i] = xn
        if store_xr is not None:
            store_xr(i, xn)
        return 0

    jax.lax.fori_loop(0, nchunks, apply_body, 0)


def _bn1d(h, gamma, beta):
    mean = h.mean(axis=0)
    var = ((h - mean) ** 2).mean(axis=0)
    return gamma * (h - mean) / jnp.sqrt(var + 1e-5) + beta


def _body(pt_ref, ax_ref, ay_ref,
          W0aT_ref, b0a_ref, g0a_ref, be0a_ref, W1aT_ref, b1a_ref,
          W0bT_ref, b0b_ref, g0b_ref, be0b_ref, W1bT_ref, b1b_ref,
          RW0_ref, Rb0_ref, Rg0_ref, Rbe0_ref,
          RW1_ref, Rb1_ref, Rg1_ref, Rbe1_ref, RW2_ref, Rb2_ref,
          out_ref, xs_ref, xr_ref, *, bc, nchunks):

    def load_a(i):
        ptc = pt_ref[pl.ds(i * bc, bc)]
        axc = ax_ref[pl.ds(i * bc, bc)]
        ayc = ay_ref[pl.ds(i * bc, bc)]
        absang = jnp.sqrt(axc * axc + ayc * ayc)
        return jnp.concatenate(
            [ptc[None], absang[None], axc[None], ayc[None]], axis=0)

    _conv_layer(load_a, W0aT_ref[...], b0a_ref[...], g0a_ref[...],
                be0a_ref[...], W1aT_ref[...], b1a_ref[...],
                xs_ref, nchunks, 1)

    def load_b(i):
        return xs_ref[i]

    def store_xr(i, xn):
        ptc = pt_ref[pl.ds(i * bc, bc)]
        wjc = ptc / ptc.sum(axis=1, keepdims=True)
        xr_c = (wjc[None] * xn[1:13]).sum(axis=2)
        xr_ref[pl.ds(i * bc, bc), :] = jnp.transpose(xr_c, (1, 0))

    _conv_layer(load_b, W0bT_ref[...], b0b_ref[...], g0b_ref[...],
                be0b_ref[...], W1bT_ref[...], b1b_ref[...],
                xs_ref, nchunks, 10, store_xr=store_xr)

    xr = xr_ref[...]
    h = jnp.dot(xr[:, :10].astype(jnp.bfloat16),
                RW0_ref[...].astype(jnp.bfloat16),
                preferred_element_type=jnp.float32) + Rb0_ref[...]
    h = _leaky(_bn1d(h, Rg0_ref[...], Rbe0_ref[...]))
    h = jnp.dot(h.astype(jnp.bfloat16), RW1_ref[...].astype(jnp.bfloat16),
                preferred_element_type=jnp.float32) + Rb1_ref[...]
    h = _leaky(_bn1d(h, Rg1_ref[...], Rbe1_ref[...]))
    h = jnp.dot(h.astype(jnp.bfloat16), RW2_ref[...].astype(jnp.bfloat16),
                preferred_element_type=jnp.float32) + Rb2_ref[...]
    out = 1.0 / (1.0 + jnp.exp(-h))
    out_ref[...] = jnp.concatenate([out, xr[:, 10:12]], axis=-1)


def kernel(pt, angles, W0a, b0a, g0a, be0a, W1a, b1a,
           W0b, b0b, g0b, be0b, W1b, b1b,
           RW0, Rb0, Rg0, Rbe0, RW1, Rb1, Rg1, Rbe1, RW2, Rb2):
    B, N = pt.shape
    bc = 32 if B % 32 == 0 else B
    nchunks = B // bc

    body = functools.partial(_body, bc=bc, nchunks=nchunks)

    return pl.pallas_call(
        body,
        out_shape=jax.ShapeDtypeStruct((B, 3), jnp.float32),
        scratch_shapes=[
            pltpu.VMEM((nchunks, 13, bc, N), jnp.float32),
            pltpu.VMEM((B, 12), jnp.float32),
        ],
    )(pt, angles[..., 0], angles[..., 1],
      W0a.T, b0a, g0a, be0a, W1a.T, b1a,
      W0b.T, b0b, g0b, be0b, W1b.T, b1b,
      RW0, Rb0, Rg0, Rbe0, RW1, Rb1, Rg1, Rbe1, RW2, Rb2)
